# baseline (speedup 1.0000x reference)
import functools

import jax
import jax.numpy as jnp
from jax import lax
from jax.experimental import pallas as pl
from jax.experimental.pallas import tpu as pltpu
from jax.experimental.pallas import tpu_sc as plsc

N = 100000
D = 128
V = 10
VP = 16
NC, NS = 2, 16
NW = NC * NS
CHUNK = 128

TCB = 2000
N2 = 40000
N1 = N - N2


@functools.lru_cache(maxsize=2)
def _build_sc(n1):
    nchunks = (n1 + CHUNK - 1) // CHUNK
    t = (nchunks + NW - 1) // NW
    span_rows = t * CHUNK

    @functools.partial(
        pl.kernel,
        out_type=jax.ShapeDtypeStruct((n1, D), jnp.float32),
        mesh=plsc.VectorSubcoreMesh(core_axis_name="c", subcore_axis_name="s"),
        scratch_types=[
            pltpu.VMEM_SHARED((V, D), jnp.float32),
            pltpu.VMEM((span_rows,), jnp.int32),
            pltpu.VMEM((CHUNK, D), jnp.float32),
            pltpu.VMEM((CHUNK, D), jnp.float32),
            pltpu.SemaphoreType.DMA,
            pltpu.SemaphoreType.DMA,
            pltpu.SemaphoreType.DMA,
            pltpu.SemaphoreType.DMA,
        ],
    )
    def _lookup(idx_hbm, tab_hbm, out_hbm, tab_sh, idx_all, rows0, rows1,
                g0, g1, w0, w1):
        cid = lax.axis_index("c")
        sid = lax.axis_index("s")
        wid = sid * NC + cid

        @pl.when(sid == 0)
        def _():
            pltpu.sync_copy(tab_hbm, tab_sh)

        plsc.subcore_barrier()

        span = jnp.minimum((wid * nchunks) // NW * CHUNK, n1 - span_rows)
        pltpu.sync_copy(idx_hbm.at[pl.ds(span, span_rows)], idx_all)

        rows = (rows0, rows1)
        gsem = (g0, g1)
        wsem = (w0, w1)

        def start_gather(i, b):
            return pltpu.async_copy(
                tab_sh.at[idx_all.at[pl.ds(i * CHUNK, CHUNK)]], rows[b], gsem[b]
            )

        gd = [start_gather(0, 0), None]
        wd = [None, None]
        for i in range(t):
            b = i & 1
            nb = 1 - b
            if i + 1 < t:
                if wd[nb] is not None:
                    wd[nb].wait()
                gd[nb] = start_gather(i + 1, nb)
            gd[b].wait()
            wd[b] = pltpu.async_copy(
                rows[b], out_hbm.at[pl.ds(span + i * CHUNK, CHUNK)], wsem[b]
            )
        for d in wd:
            if d is not None:
                d.wait()

    return _lookup


@functools.lru_cache(maxsize=2)
def _build_tc(n2):
    nb = n2 // TCB

    def body(idx_ref, tab_ref, out_ref):
        idx = idx_ref[0, 0, :]
        onehot = (idx[:, None] == lax.broadcasted_iota(jnp.int32, (1, VP), 1)
                  ).astype(jnp.float32)
        out_ref[...] = jnp.dot(
            onehot, tab_ref[...], preferred_element_type=jnp.float32
        )

    return pl.pallas_call(
        body,
        grid=(nb,),
        in_specs=[
            pl.BlockSpec((1, 1, TCB), lambda i: (i, 0, 0)),
            pl.BlockSpec((VP, D), lambda i: (0, 0)),
        ],
        out_specs=pl.BlockSpec((TCB, D), lambda i: (i, 0)),
        out_shape=jax.ShapeDtypeStruct((n2, D), jnp.float32),
    )


def kernel(atomic_num, embedding):
    idx = atomic_num.astype(jnp.int32)
    out_sc = _build_sc(N1)(idx[:N1], embedding)
    tab_pad = jnp.pad(embedding, ((0, VP - V), (0, 0)))
    idx_tc = idx[N1:].reshape(N2 // TCB, 1, TCB)
    out_tc = _build_tc(N2)(idx_tc, tab_pad)
    return jnp.concatenate([out_sc, out_tc], axis=0)


# speedup vs baseline: 1.6678x; 1.6678x over previous
import functools

import jax
import jax.numpy as jnp
from jax import lax
from jax.experimental import pallas as pl
from jax.experimental.pallas import tpu as pltpu
from jax.experimental.pallas import tpu_sc as plsc

N = 100000
D = 128
V = 10
NC, NS = 2, 16
NW = NC * NS
CHUNK = 128
NCHUNKS = (N + CHUNK - 1) // CHUNK
T = (NCHUNKS + NW - 1) // NW
SPAN = T * CHUNK
L = 16


@functools.lru_cache(maxsize=1)
def _build():
    @functools.partial(
        pl.kernel,
        out_type=jax.ShapeDtypeStruct((N, D), jnp.float32),
        mesh=plsc.VectorSubcoreMesh(core_axis_name="c", subcore_axis_name="s"),
        scratch_types=[
            pltpu.VMEM_SHARED((NS * V, D), jnp.float32),
            pltpu.VMEM((SPAN,), jnp.int32),
            pltpu.VMEM((CHUNK, D), jnp.float32),
            pltpu.VMEM((CHUNK, D), jnp.float32),
            pltpu.SemaphoreType.DMA,
            pltpu.SemaphoreType.DMA,
            pltpu.SemaphoreType.DMA,
            pltpu.SemaphoreType.DMA,
        ],
    )
    def _lookup(idx_hbm, tab_hbm, out_hbm, tab_sh, idx_all, rows0, rows1,
                g0, g1, w0, w1):
        cid = lax.axis_index("c")
        sid = lax.axis_index("s")
        wid = sid * NC + cid

        pltpu.sync_copy(tab_hbm, tab_sh.at[pl.ds(sid * V, V)])

        span = jnp.minimum((wid * NCHUNKS) // NW * CHUNK, N - SPAN)
        pltpu.sync_copy(idx_hbm.at[pl.ds(span, SPAN)], idx_all)

        base = sid * V
        for j in range(SPAN // L):
            sl = pl.ds(j * L, L)
            idx_all[sl] = idx_all[sl] + base

        plsc.subcore_barrier()

        rows = (rows0, rows1)
        gsem = (g0, g1)
        wsem = (w0, w1)

        def start_gather(i, b):
            return pltpu.async_copy(
                tab_sh.at[idx_all.at[pl.ds(i * CHUNK, CHUNK)]], rows[b], gsem[b]
            )

        gd = [start_gather(0, 0), None]
        wd = [None, None]
        for i in range(T):
            b = i & 1
            nb = 1 - b
            if i + 1 < T:
                if wd[nb] is not None:
                    wd[nb].wait()
                gd[nb] = start_gather(i + 1, nb)
            gd[b].wait()
            wd[b] = pltpu.async_copy(
                rows[b], out_hbm.at[pl.ds(span + i * CHUNK, CHUNK)], wsem[b]
            )
        for d in wd:
            if d is not None:
                d.wait()

    return _lookup


def kernel(atomic_num, embedding):
    idx = atomic_num.astype(jnp.int32)
    return _build()(idx, embedding)


# speedup vs baseline: 1.6833x; 1.0093x over previous
import functools

import jax
import jax.numpy as jnp
from jax import lax
from jax.experimental import pallas as pl
from jax.experimental.pallas import tpu as pltpu
from jax.experimental.pallas import tpu_sc as plsc

N = 100000
D = 128
V = 10
NC, NS = 2, 16
NW = NC * NS
CHUNK = 128
NCHUNKS = (N + CHUNK - 1) // CHUNK
T = (NCHUNKS + NW - 1) // NW
SPAN = T * CHUNK
NBUF = 4


@functools.lru_cache(maxsize=1)
def _build():
    @functools.partial(
        pl.kernel,
        out_type=jax.ShapeDtypeStruct((N, D), jnp.float32),
        mesh=plsc.VectorSubcoreMesh(core_axis_name="c", subcore_axis_name="s"),
        scratch_types=[
            pltpu.VMEM_SHARED((V, D), jnp.float32),
            pltpu.VMEM((SPAN,), jnp.int32),
        ]
        + [pltpu.VMEM((CHUNK, D), jnp.float32) for _ in range(NBUF)]
        + [pltpu.SemaphoreType.DMA for _ in range(2 * NBUF)],
    )
    def _lookup(idx_hbm, tab_hbm, out_hbm, tab_sh, idx_all, *bufs_and_sems):
        rows = bufs_and_sems[:NBUF]
        gsem = bufs_and_sems[NBUF:2 * NBUF]
        wsem = bufs_and_sems[2 * NBUF:]

        cid = lax.axis_index("c")
        sid = lax.axis_index("s")
        wid = sid * NC + cid

        @pl.when(sid == 0)
        def _():
            pltpu.sync_copy(tab_hbm, tab_sh)

        plsc.subcore_barrier()

        span = jnp.minimum((wid * NCHUNKS) // NW * CHUNK, N - SPAN)
        pltpu.sync_copy(idx_hbm.at[pl.ds(span, SPAN)], idx_all)

        def start_gather(i):
            b = i % NBUF
            return pltpu.async_copy(
                tab_sh.at[idx_all.at[pl.ds(i * CHUNK, CHUNK)]], rows[b], gsem[b]
            )

        def start_write(i):
            b = i % NBUF
            return pltpu.async_copy(
                rows[b], out_hbm.at[pl.ds(span + i * CHUNK, CHUNK)], wsem[b]
            )

        gd = [None] * T
        wd = [None] * T
        for j in range(min(NBUF, T)):
            gd[j] = start_gather(j)
        for i in range(T):
            if i >= 2 and i + 2 < T:
                wd[i - 2].wait()
                gd[i + 2] = start_gather(i + 2)
            gd[i].wait()
            wd[i] = start_write(i)
        for i in range(max(0, T - NBUF), T):
            if wd[i] is not None:
                wd[i].wait()

    return _lookup


def kernel(atomic_num, embedding):
    idx = atomic_num.astype(jnp.int32)
    return _build()(idx, embedding)


# speedup vs baseline: 1.7159x; 1.0194x over previous
import functools

import jax
import jax.numpy as jnp
from jax import lax
from jax.experimental import pallas as pl
from jax.experimental.pallas import tpu as pltpu
from jax.experimental.pallas import tpu_sc as plsc

N = 100000
D = 128
V = 10
NC, NS = 2, 16
NW = NC * NS
CHUNK = 128
NCHUNKS = (N + CHUNK - 1) // CHUNK
T = (NCHUNKS + NW - 1) // NW
SPAN = T * CHUNK
NBUF = 4


@functools.lru_cache(maxsize=1)
def _build():
    @functools.partial(
        pl.kernel,
        out_type=jax.ShapeDtypeStruct((N, D), jnp.float32),
        mesh=plsc.VectorSubcoreMesh(core_axis_name="c", subcore_axis_name="s"),
        scratch_types=[
            pltpu.VMEM_SHARED((V, D), jnp.float32),
            pltpu.VMEM((SPAN,), jnp.int32),
        ]
        + [pltpu.VMEM((CHUNK, D), jnp.float32) for _ in range(NBUF)]
        + [pltpu.SemaphoreType.DMA for _ in range(2 * NBUF + 2)],
    )
    def _lookup(idx_hbm, tab_hbm, out_hbm, tab_sh, idx_all, *bufs_and_sems):
        rows = bufs_and_sems[:NBUF]
        gsem = bufs_and_sems[NBUF:2 * NBUF]
        wsem = bufs_and_sems[2 * NBUF:3 * NBUF]
        isem0, isem1 = bufs_and_sems[3 * NBUF:]

        cid = lax.axis_index("c")
        sid = lax.axis_index("s")
        wid = sid * NC + cid

        span = jnp.minimum((wid * NCHUNKS) // NW * CHUNK, N - SPAN)

        idx0 = pltpu.async_copy(
            idx_hbm.at[pl.ds(span, CHUNK)], idx_all.at[pl.ds(0, CHUNK)], isem0
        )
        idx_rest = pltpu.async_copy(
            idx_hbm.at[pl.ds(span + CHUNK, SPAN - CHUNK)],
            idx_all.at[pl.ds(CHUNK, SPAN - CHUNK)],
            isem1,
        )

        @pl.when(sid == 0)
        def _():
            pltpu.sync_copy(tab_hbm, tab_sh)

        plsc.subcore_barrier()

        def start_gather(i):
            b = i % NBUF
            return pltpu.async_copy(
                tab_sh.at[idx_all.at[pl.ds(i * CHUNK, CHUNK)]], rows[b], gsem[b]
            )

        def start_write(i):
            b = i % NBUF
            return pltpu.async_copy(
                rows[b], out_hbm.at[pl.ds(span + i * CHUNK, CHUNK)], wsem[b]
            )

        gd = [None] * T
        wd = [None] * T
        idx0.wait()
        gd[0] = start_gather(0)
        idx_rest.wait()
        for j in range(1, min(NBUF, T)):
            gd[j] = start_gather(j)
        for i in range(T):
            if i >= 2 and i + 2 < T:
                wd[i - 2].wait()
                gd[i + 2] = start_gather(i + 2)
            gd[i].wait()
            wd[i] = start_write(i)
        for i in range(max(0, T - NBUF), T):
            if wd[i] is not None:
                wd[i].wait()

    return _lookup


def kernel(atomic_num, embedding):
    idx = atomic_num.astype(jnp.int32)
    return _build()(idx, embedding)
